# baseline (speedup 1.0000x reference)
import functools

import jax
import jax.numpy as jnp
from jax import lax
from jax.experimental import pallas as pl
from jax.experimental.pallas import tpu as pltpu
from jax.experimental.pallas import tpu_sc as plsc

BATCH = 16384
HALF = BATCH // 2
NUM_SUBCORES = 16
B_PER_W = HALF // NUM_SUBCORES


def _build(name):
    mesh = plsc.VectorSubcoreMesh(
        core_axis_name="c", subcore_axis_name="s", num_cores=1
    )

    @functools.partial(
        pl.kernel,
        mesh=mesh,
        out_type=jax.ShapeDtypeStruct((HALF,), jnp.float32),
        scratch_types=[
            pltpu.VMEM((B_PER_W,), jnp.int32),
            pltpu.VMEM((B_PER_W,), jnp.float32),
            pltpu.SemaphoreType.DMA,
            pltpu.SemaphoreType.DMA,
            pltpu.SemaphoreType.DMA,
        ],
        name=name,
    )
    def gather_kernel(table_hbm, idx_hbm, out_hbm, idx_v, rows_v,
                      sem_i, sem_g, sem_o):
        wid = lax.axis_index("s")
        base = wid * B_PER_W
        pltpu.async_copy(idx_hbm.at[pl.ds(base, B_PER_W)], idx_v,
                         sem_i).wait()
        pltpu.async_copy(table_hbm.at[idx_v], rows_v, sem_g).wait()
        pltpu.async_copy(rows_v, out_hbm.at[pl.ds(base, B_PER_W)],
                         sem_o).wait()

    return gather_kernel


_gather_lo = _build("gather_lo")
_gather_hi = _build("gather_hi")


@jax.jit
def kernel(data, ratio_target_lookup, targets):
    del data
    lo = _gather_lo(ratio_target_lookup, targets[:HALF])
    hi = _gather_hi(ratio_target_lookup, targets[HALF:])
    return jnp.concatenate([lo, hi])


# speedup vs baseline: 1.3110x; 1.3110x over previous
import functools

import jax
import jax.numpy as jnp
from jax import lax
from jax.experimental import pallas as pl
from jax.experimental.pallas import tpu as pltpu
from jax.experimental.pallas import tpu_sc as plsc

BATCH = 16384
NUM_SUBCORES = 16
B_PER_W = BATCH // NUM_SUBCORES


def _build():
    mesh = plsc.VectorSubcoreMesh(
        core_axis_name="c", subcore_axis_name="s", num_cores=1
    )

    @functools.partial(
        pl.kernel,
        mesh=mesh,
        out_type=jax.ShapeDtypeStruct((BATCH,), jnp.float32),
        scratch_types=[
            pltpu.VMEM((B_PER_W,), jnp.int32),
            pltpu.VMEM((B_PER_W,), jnp.float32),
            pltpu.SemaphoreType.DMA,
        ],
    )
    def gather_kernel(table_hbm, idx_hbm, out_hbm, idx_v, rows_v, sem):
        wid = lax.axis_index("s")
        base = wid * B_PER_W
        pltpu.sync_copy(idx_hbm.at[pl.ds(base, B_PER_W)], idx_v)
        pltpu.async_copy(table_hbm.at[idx_v], rows_v, sem).wait()
        pltpu.sync_copy(rows_v, out_hbm.at[pl.ds(base, B_PER_W)])

    return gather_kernel


_gather = _build()


@jax.jit
def kernel(data, ratio_target_lookup, targets):
    del data
    return _gather(ratio_target_lookup, targets)
